# baseline (speedup 1.0000x reference)
import functools

import jax
import jax.numpy as jnp
from jax import lax
from jax.experimental import pallas as pl
from jax.experimental.pallas import tpu as pltpu
from jax.experimental.pallas import tpu_sc as plsc

_B, _N, _D = 4, 4096, 2048
_NW = 32
_ROWS_PW = (_B * _N) // _NW
_SPB = _N // _ROWS_PW
_CHUNK = 32
_NCH = _ROWS_PW // _CHUNK


def _body(x_hbm, keep_hbm, tok_hbm, out_hbm, keep_v, tok_tile, sem):
    cid = lax.axis_index("c")
    sid = lax.axis_index("s")
    wid = sid * 2 + cid

    b = wid // _SPB
    n_base = (wid % _SPB) * _ROWS_PW
    row_base = wid * _ROWS_PW

    pltpu.sync_copy(keep_hbm, keep_v.at[pl.ds(0, _B)])
    kv = keep_v[...]
    kb = kv[0]
    for i in range(1, _B):
        kb = jnp.where(b == i, kv[i], kb)

    for i in range(_CHUNK):
        pltpu.sync_copy(tok_hbm, tok_tile.at[i])

    def chunk_body(c, carry):
        n0 = n_base + c * _CHUNK
        r0 = row_base + c * _CHUNK

        @pl.when(n0 + _CHUNK <= kb)
        def _copy_full():
            pltpu.sync_copy(x_hbm.at[pl.ds(r0, _CHUNK)],
                            out_hbm.at[pl.ds(r0, _CHUNK)])

        @pl.when(n0 >= kb)
        def _tok_full():
            pltpu.sync_copy(tok_tile, out_hbm.at[pl.ds(r0, _CHUNK)])

        @pl.when(jnp.logical_and(n0 < kb, n0 + _CHUNK > kb))
        def _straddle():
            def row_body(j, carry2):
                @pl.when(n0 + j < kb)
                def _r_copy():
                    pltpu.sync_copy(x_hbm.at[pl.ds(r0 + j, 1)],
                                    out_hbm.at[pl.ds(r0 + j, 1)])

                @pl.when(n0 + j >= kb)
                def _r_tok():
                    pltpu.sync_copy(tok_tile.at[pl.ds(0, 1)],
                                    out_hbm.at[pl.ds(r0 + j, 1)])
                return carry2
            lax.fori_loop(0, _CHUNK, row_body, 0)
        return carry

    lax.fori_loop(0, _NCH, chunk_body, 0)


def kernel(x, eval_keep_k, dropout_mask_token):
    Bx, Nx, Dx = x.shape
    x2 = x.reshape(Bx * Nx, Dx)
    mesh = plsc.VectorSubcoreMesh(core_axis_name="c", subcore_axis_name="s")
    run = pl.kernel(
        _body,
        out_type=jax.ShapeDtypeStruct((Bx * Nx, Dx), x.dtype),
        mesh=mesh,
        scratch_types=[
            pltpu.VMEM((16,), jnp.int32),
            pltpu.VMEM((_CHUNK, _D), jnp.float32),
            pltpu.SemaphoreType.DMA,
        ],
    )
    out = run(x2, eval_keep_k.astype(jnp.int32), dropout_mask_token)
    return out.reshape(Bx, Nx, Dx)
